# Initial kernel scaffold; baseline (speedup 1.0000x reference)
import jax
import jax.numpy as jnp
from jax.experimental import pallas as pl


def kernel(hidden_states, topk_ids, topk_weights, gate_a, gate_b, up_a, up_b, down_a, down_b, weight_indices, seq_lens, lora_ranks, scalings, base_gate_up_weight, base_down_weight):
    raise NotImplementedError("write your pallas kernel here")


# speedup vs baseline: 1.9244x; 1.9244x over previous
import functools

import jax
import jax.numpy as jnp
from jax import lax
from jax.experimental import pallas as pl
from jax.experimental.pallas import tpu as pltpu

_TM = 256


def _moe_body(ids_ref, tw_ref, x_ref, wgu_ref, wd_ref, ga_ref, gb_ref,
              ua_ref, ub_ref, da_ref, db_ref, scal_ref, out_ref, acc_ref,
              *, n_experts, inter, tm):
    e = pl.program_id(0)
    t = pl.program_id(1)
    s = scal_ref[0, 0]

    x = x_ref[...]
    wgu = wgu_ref[0]
    gu = jnp.dot(x, wgu, preferred_element_type=jnp.float32)

    xga = lax.dot_general(x, ga_ref[0], (((1,), (1,)), ((), ())),
                          preferred_element_type=jnp.float32)
    lg = lax.dot_general(xga, gb_ref[0], (((1,), (1,)), ((), ())),
                         preferred_element_type=jnp.float32)
    xua = lax.dot_general(x, ua_ref[0], (((1,), (1,)), ((), ())),
                          preferred_element_type=jnp.float32)
    lu = lax.dot_general(xua, ub_ref[0], (((1,), (1,)), ((), ())),
                         preferred_element_type=jnp.float32)

    gate = gu[:, :inter] + s * lg
    up = gu[:, inter:] + s * lu
    act = jax.nn.silu(gate) * up

    y = jnp.dot(act, wd_ref[0], preferred_element_type=jnp.float32)
    ada = lax.dot_general(act, da_ref[0], (((1,), (1,)), ((), ())),
                          preferred_element_type=jnp.float32)
    ld = lax.dot_general(ada, db_ref[0], (((1,), (1,)), ((), ())),
                         preferred_element_type=jnp.float32)
    y = y + s * ld

    ids = ids_ref[...]
    tw = tw_ref[...]
    w = jnp.sum(jnp.where(ids == e, tw, jnp.zeros_like(tw)), axis=1)
    contrib = y * w[:, None]

    sl = pl.ds(t * tm, tm)

    @pl.when(e == 0)
    def _init():
        acc_ref[sl, :] = contrib

    @pl.when(e > 0)
    def _accum():
        acc_ref[sl, :] += contrib

    @pl.when(e == n_experts - 1)
    def _emit():
        out_ref[...] = acc_ref[sl, :]


def kernel(hidden_states, topk_ids, topk_weights, gate_a, gate_b, up_a, up_b,
           down_a, down_b, weight_indices, seq_lens, lora_ranks, scalings,
           base_gate_up_weight, base_down_weight):
    T, H = hidden_states.shape
    E, _, I2 = base_gate_up_weight.shape
    inter = I2 // 2
    R = gate_a.shape[2]
    K = topk_ids.shape[1]
    tm = _TM
    nt = T // tm

    adapter = weight_indices[0]
    ga = lax.dynamic_index_in_dim(gate_a, adapter, 0, keepdims=False)
    gb = lax.dynamic_index_in_dim(gate_b, adapter, 0, keepdims=False)
    ua = lax.dynamic_index_in_dim(up_a, adapter, 0, keepdims=False)
    ub = lax.dynamic_index_in_dim(up_b, adapter, 0, keepdims=False)
    da = lax.dynamic_index_in_dim(down_a, adapter, 0, keepdims=False)
    db = lax.dynamic_index_in_dim(down_b, adapter, 0, keepdims=False)
    scal = scalings[adapter].reshape(1, 1).astype(jnp.float32)

    body = functools.partial(_moe_body, n_experts=E, inter=inter, tm=tm)

    out = pl.pallas_call(
        body,
        grid=(E, nt),
        in_specs=[
            pl.BlockSpec((tm, K), lambda e, t: (t, 0)),
            pl.BlockSpec((tm, K), lambda e, t: (t, 0)),
            pl.BlockSpec((tm, H), lambda e, t: (t, 0)),
            pl.BlockSpec((1, H, I2), lambda e, t: (e, 0, 0)),
            pl.BlockSpec((1, inter, H), lambda e, t: (e, 0, 0)),
            pl.BlockSpec((1, R, H), lambda e, t: (e, 0, 0)),
            pl.BlockSpec((1, inter, R), lambda e, t: (e, 0, 0)),
            pl.BlockSpec((1, R, H), lambda e, t: (e, 0, 0)),
            pl.BlockSpec((1, inter, R), lambda e, t: (e, 0, 0)),
            pl.BlockSpec((1, R, inter), lambda e, t: (e, 0, 0)),
            pl.BlockSpec((1, H, R), lambda e, t: (e, 0, 0)),
            pl.BlockSpec(memory_space=pltpu.SMEM),
        ],
        out_specs=pl.BlockSpec((tm, H), lambda e, t: (t, 0)),
        out_shape=jax.ShapeDtypeStruct((T, H), jnp.float32),
        scratch_shapes=[pltpu.VMEM((T, H), jnp.float32)],
    )(topk_ids, topk_weights, hidden_states.astype(jnp.float32),
      base_gate_up_weight, base_down_weight, ga, gb, ua, ub, da, db, scal)

    return out.astype(hidden_states.dtype)
